# baseline (speedup 1.0000x reference)
import functools

import jax
import jax.numpy as jnp
from jax import lax
from jax.experimental import pallas as pl
from jax.experimental.pallas import tpu as pltpu
from jax.experimental.pallas import tpu_sc as plsc

_N = 10000
_NPAD = 10240
_E = 320000
_NC, _NS = 2, 16
_CH = 128
_NCHF = 160
_NBUF = 2
_EPAD = _NCHF * _CH * _NS
_STRIPE = _NPAD // _NS
_DH = 128
_DH2 = _DH // 2
_DP3 = 64
_DP32 = _DP3 // 2


def _sc_mesh():
    return plsc.VectorSubcoreMesh(
        core_axis_name="c", subcore_axis_name="s",
        num_cores=_NC, num_subcores=_NS)


def _make_deg():
    nch = _NCHF // _NC

    @functools.partial(
        pl.kernel,
        out_type=jax.ShapeDtypeStruct((_NC, _NPAD), jnp.float32),
        mesh=_sc_mesh(),
        scratch_types=[
            pltpu.VMEM((nch, _CH), jnp.int32),
            pltpu.VMEM((_STRIPE,), jnp.float32),
            pltpu.VMEM((_CH,), jnp.float32),
            pltpu.VMEM_SHARED((_NPAD,), jnp.float32),
        ],
    )
    def deg_kernel(dst_hbm, out_hbm, dst_v, zbuf, ones_v, acc):
        c = lax.axis_index("c")
        s = lax.axis_index("s")
        zero16 = jnp.zeros((16,), jnp.float32)
        one16 = jnp.ones((16,), jnp.float32)
        for i in range(_STRIPE // 16):
            zbuf[pl.ds(i * 16, 16)] = zero16
        for i in range(_CH // 16):
            ones_v[pl.ds(i * 16, 16)] = one16
        pltpu.sync_copy(zbuf, acc.at[pl.ds(s * _STRIPE, _STRIPE)])
        plsc.subcore_barrier()
        pltpu.sync_copy(dst_hbm.at[s, pl.ds(c * nch, nch)], dst_v)

        def body(j, carry):
            pltpu.sync_copy(ones_v, acc.at[dst_v.at[j]], add=True)
            return carry

        lax.fori_loop(0, nch, body, 0)
        plsc.subcore_barrier()
        pltpu.sync_copy(acc.at[pl.ds(s * _STRIPE, _STRIPE)],
                        out_hbm.at[c, pl.ds(s * _STRIPE, _STRIPE)])

    return deg_kernel


def _make_agg(dh):
    @functools.partial(
        pl.kernel,
        out_type=jax.ShapeDtypeStruct((_NC, _NPAD, dh), jnp.float32),
        mesh=_sc_mesh(),
        scratch_types=[
            pltpu.VMEM((_NCHF, _CH), jnp.int32),
            pltpu.VMEM((_NCHF, _CH), jnp.int32),
            pltpu.VMEM((_NBUF, _CH, dh), jnp.float32),
            pltpu.VMEM_SHARED((_NPAD, dh), jnp.float32),
        ] + [pltpu.SemaphoreType.DMA] * (2 * _NBUF),
        compiler_params=pltpu.CompilerParams(use_tc_tiling_on_sc=False),
    )
    def agg_kernel(hs_hbm, src_hbm, dst_hbm, out_hbm, src_v, dst_v, rows_v,
                   acc, *sems):
        gsem, ssem = sems[:_NBUF], sems[_NBUF:]
        c = lax.axis_index("c")
        s = lax.axis_index("s")
        zero16 = jnp.zeros((16,), jnp.float32)

        def zrow(r, carry):
            for k in range(dh // 16):
                rows_v[0, r, pl.ds(k * 16, 16)] = zero16
            return carry

        lax.fori_loop(0, _CH, zrow, 0)
        for k in range(_STRIPE // _CH):
            pltpu.sync_copy(rows_v.at[0],
                            acc.at[pl.ds(s * _STRIPE + k * _CH, _CH)])
        plsc.subcore_barrier()
        pltpu.sync_copy(src_hbm.at[s], src_v)
        pltpu.sync_copy(dst_hbm.at[s], dst_v)

        for b in range(_NBUF):
            pltpu.async_copy(hs_hbm.at[c].at[src_v.at[b]], rows_v.at[b],
                             gsem[b])

        def group(g, carry):
            base = g * _NBUF
            for b in range(_NBUF):
                pltpu.make_async_copy(hs_hbm.at[c].at[src_v.at[base + b]],
                                      rows_v.at[b], gsem[b]).wait()
                pltpu.async_copy(rows_v.at[b], acc.at[dst_v.at[base + b]],
                                 ssem[b], add=True)
            for b in range(_NBUF):
                pltpu.make_async_copy(rows_v.at[b],
                                      acc.at[dst_v.at[base + b]],
                                      ssem[b]).wait()
                nxt = base + _NBUF + b

                @pl.when(nxt < _NCHF)
                def _():
                    pltpu.async_copy(hs_hbm.at[c].at[src_v.at[nxt]],
                                     rows_v.at[b], gsem[b])
            return carry

        lax.fori_loop(0, _NCHF // _NBUF, group, 0)
        plsc.subcore_barrier()
        pltpu.sync_copy(acc.at[pl.ds(s * _STRIPE, _STRIPE)],
                        out_hbm.at[c, pl.ds(s * _STRIPE, _STRIPE)])

    return agg_kernel


def _split(h, dh):
    return jnp.stack([h[:, :dh], h[:, dh:2 * dh]], axis=0)


def _tc_a(x_ref, w_ref, degt_ref, hs_ref, dinv_ref):
    deg = jnp.sum(degt_ref[...], axis=1, keepdims=True) + 1.0
    dinv = lax.rsqrt(deg)
    h = jnp.dot(x_ref[...], w_ref[...], preferred_element_type=jnp.float32)
    hs_ref[...] = _split(h * dinv, _DH2)
    dinv_ref[...] = dinv


def _tc_b(p_ref, hs_ref, dinv_ref, b_ref, g_ref, be_ref, w_ref, out_ref, dh):
    dinv = dinv_ref[...]
    agg = jnp.concatenate([p_ref[0] + hs_ref[0], p_ref[1] + hs_ref[1]],
                          axis=-1)
    t = agg * dinv + b_ref[...]
    rows = lax.broadcasted_iota(jnp.int32, t.shape, 0)
    t = jnp.where(rows < _N, t, 0.0)
    m = jnp.sum(t, axis=0, keepdims=True) * (1.0 / _N)
    v = jnp.sum(t * t, axis=0, keepdims=True) * (1.0 / _N) - m * m
    a = (t - m) * lax.rsqrt(v + 1e-5) * g_ref[...] + be_ref[...]
    a = jnp.maximum(a, 0.0)
    a = jnp.where(rows < _N, a, 0.0)
    h = jnp.dot(a, w_ref[...], preferred_element_type=jnp.float32)
    out_ref[...] = _split(h * dinv, dh)


def _tc_c(p_ref, hs_ref, dinv_ref, b_ref, out_ref):
    agg = jnp.concatenate([p_ref[0] + hs_ref[0], p_ref[1] + hs_ref[1]],
                          axis=-1)
    t = agg * dinv_ref[...]
    t = t[:_N, :40] + b_ref[...]
    mx = jnp.max(t, axis=1, keepdims=True)
    lse = jnp.log(jnp.sum(jnp.exp(t - mx), axis=1, keepdims=True)) + mx
    out_ref[...] = t - lse


def kernel(x, W1, b1, g1, be1, W2, b2, g2, be2, W3, b3, edge_index):
    fill = jnp.full((_EPAD - _E,), _N, jnp.int32)
    src3 = jnp.concatenate([edge_index[0], fill]).reshape(_NS, _NCHF, _CH)
    dst3 = jnp.concatenate([edge_index[1], fill]).reshape(_NS, _NCHF, _CH)
    x_pad = jnp.pad(x, ((0, _NPAD - _N), (0, 0)))
    w3p = jnp.pad(W3, ((0, 0), (0, _DP3 - 40)))

    degp = _make_deg()(dst3)
    degt = degp.T

    f32 = jnp.float32
    hs1, dinv = pl.pallas_call(
        _tc_a,
        out_shape=[jax.ShapeDtypeStruct((_NC, _NPAD, _DH2), f32),
                   jax.ShapeDtypeStruct((_NPAD, 1), f32)],
    )(x_pad, W1, degt)

    agg128 = _make_agg(_DH2)
    p1 = agg128(hs1, src3, dst3)
    hs2 = pl.pallas_call(
        functools.partial(_tc_b, dh=_DH2),
        out_shape=jax.ShapeDtypeStruct((_NC, _NPAD, _DH2), f32),
    )(p1, hs1, dinv, b1, g1, be1, W2)

    p2 = agg128(hs2, src3, dst3)
    hs3 = pl.pallas_call(
        functools.partial(_tc_b, dh=_DP32),
        out_shape=jax.ShapeDtypeStruct((_NC, _NPAD, _DP32), f32),
    )(p2, hs2, dinv, b2, g2, be2, w3p)

    p3 = _make_agg(_DP32)(hs3, src3, dst3)
    out = pl.pallas_call(
        _tc_c, out_shape=jax.ShapeDtypeStruct((_N, 40), f32),
    )(p3, hs3, dinv, b3)
    return out


# speedup vs baseline: 1.7739x; 1.7739x over previous
import functools

import jax
import jax.numpy as jnp
from jax import lax
from jax.experimental import pallas as pl
from jax.experimental.pallas import tpu as pltpu
from jax.experimental.pallas import tpu_sc as plsc

_N = 10000
_E = 320000
_NC, _NS = 2, 16
_CH = 125
_NCH = 80
_STRIPE = _N // _NS
_DH = 128
_DP3 = 48


def _sc_mesh():
    return plsc.VectorSubcoreMesh(
        core_axis_name="c", subcore_axis_name="s",
        num_cores=_NC, num_subcores=_NS)


_DEGPAD = 10240
_DSTRIPE = _DEGPAD // _NS


def _make_deg():
    @functools.partial(
        pl.kernel,
        out_type=jax.ShapeDtypeStruct((_NC, _DEGPAD), jnp.float32),
        mesh=_sc_mesh(),
        scratch_types=[
            pltpu.VMEM((_NCH, _CH), jnp.int32),
            pltpu.VMEM((_DSTRIPE,), jnp.float32),
            pltpu.VMEM((_CH,), jnp.float32),
            pltpu.VMEM_SHARED((_DEGPAD,), jnp.float32),
        ],
    )
    def deg_kernel(dst_hbm, out_hbm, dst_v, zbuf, ones_v, acc):
        c = lax.axis_index("c")
        s = lax.axis_index("s")
        zero16 = jnp.zeros((16,), jnp.float32)
        one16 = jnp.ones((16,), jnp.float32)
        for i in range(0, _DSTRIPE, 16):
            zbuf[pl.ds(i, 16)] = zero16
        for i in range(0, _CH - 15, 16):
            ones_v[pl.ds(i, 16)] = one16
        ones_v[pl.ds(_CH - 16, 16)] = one16
        pltpu.sync_copy(zbuf, acc.at[pl.ds(s * _DSTRIPE, _DSTRIPE)])
        plsc.subcore_barrier()
        pltpu.sync_copy(dst_hbm.at[c, s], dst_v)

        def body(j, carry):
            pltpu.sync_copy(ones_v, acc.at[dst_v.at[j]], add=True)
            return carry

        lax.fori_loop(0, _NCH, body, 0)
        plsc.subcore_barrier()
        pltpu.sync_copy(acc.at[pl.ds(s * _DSTRIPE, _DSTRIPE)],
                        out_hbm.at[c, pl.ds(s * _DSTRIPE, _DSTRIPE)])

    return deg_kernel


def _make_agg(d):
    @functools.partial(
        pl.kernel,
        out_type=jax.ShapeDtypeStruct((_NC, _N, d), jnp.float32),
        mesh=_sc_mesh(),
        scratch_types=[
            pltpu.VMEM((_NCH, _CH), jnp.int32),
            pltpu.VMEM((_NCH, _CH), jnp.int32),
            pltpu.VMEM((_CH, d), jnp.float32),
            pltpu.VMEM_SHARED((_N, d), jnp.float32),
        ],
        compiler_params=pltpu.CompilerParams(use_tc_tiling_on_sc=False),
    )
    def agg_kernel(hs_hbm, src_hbm, dst_hbm, out_hbm, src_v, dst_v, rows_v,
                   acc):
        c = lax.axis_index("c")
        s = lax.axis_index("s")
        zero16 = jnp.zeros((16,), jnp.float32)

        def zrow(r, carry):
            for k in range(d // 16):
                rows_v[r, pl.ds(k * 16, 16)] = zero16
            return carry

        lax.fori_loop(0, _CH, zrow, 0)
        for o in range(0, _STRIPE, _CH):
            pltpu.sync_copy(rows_v, acc.at[pl.ds(s * _STRIPE + o, _CH)])
        plsc.subcore_barrier()
        pltpu.sync_copy(src_hbm.at[c, s], src_v)
        pltpu.sync_copy(dst_hbm.at[c, s], dst_v)

        def body(j, carry):
            pltpu.sync_copy(hs_hbm.at[src_v.at[j]], rows_v)
            pltpu.sync_copy(rows_v, acc.at[dst_v.at[j]], add=True)
            return carry

        lax.fori_loop(0, _NCH, body, 0)
        plsc.subcore_barrier()
        pltpu.sync_copy(acc.at[pl.ds(s * _STRIPE, _STRIPE)],
                        out_hbm.at[c, pl.ds(s * _STRIPE, _STRIPE)])

    return agg_kernel


def _tc_a(x_ref, w_ref, degt_ref, hs_ref, dinv_ref):
    deg = jnp.sum(degt_ref[...], axis=1, keepdims=True)[:_N] + 1.0
    dinv = lax.rsqrt(deg)
    h = jnp.dot(x_ref[...], w_ref[...], preferred_element_type=jnp.float32)
    hs_ref[...] = h * dinv
    dinv_ref[...] = dinv


def _tc_b(p_ref, hs_ref, dinv_ref, b_ref, g_ref, be_ref, w_ref, out_ref):
    dinv = dinv_ref[...]
    t = (p_ref[0] + p_ref[1] + hs_ref[...]) * dinv + b_ref[...]
    m = jnp.sum(t, axis=0, keepdims=True) * (1.0 / _N)
    v = jnp.sum(t * t, axis=0, keepdims=True) * (1.0 / _N) - m * m
    a = (t - m) * lax.rsqrt(v + 1e-5) * g_ref[...] + be_ref[...]
    a = jnp.maximum(a, 0.0)
    h = jnp.dot(a, w_ref[...], preferred_element_type=jnp.float32)
    out_ref[...] = h * dinv


def _tc_c(p_ref, hs_ref, dinv_ref, b_ref, out_ref):
    t = (p_ref[0] + p_ref[1] + hs_ref[...]) * dinv_ref[...]
    t = t[:, :40] + b_ref[...]
    mx = jnp.max(t, axis=1, keepdims=True)
    lse = jnp.log(jnp.sum(jnp.exp(t - mx), axis=1, keepdims=True)) + mx
    out_ref[...] = t - lse


def kernel(x, W1, b1, g1, be1, W2, b2, g2, be2, W3, b3, edge_index):
    e5 = edge_index.reshape(2, _NC, _NS, _NCH, _CH)
    src4, dst4 = e5[0], e5[1]
    w3p = jnp.pad(W3, ((0, 0), (0, _DP3 - 40)))

    degp = _make_deg()(dst4)
    degt = degp.T

    f32 = jnp.float32
    hs1, dinv = pl.pallas_call(
        _tc_a,
        out_shape=[jax.ShapeDtypeStruct((_N, _DH), f32),
                   jax.ShapeDtypeStruct((_N, 1), f32)],
    )(x, W1, degt)

    agg128 = _make_agg(_DH)
    p1 = agg128(hs1, src4, dst4)
    hs2 = pl.pallas_call(
        _tc_b, out_shape=jax.ShapeDtypeStruct((_N, _DH), f32),
    )(p1, hs1, dinv, b1, g1, be1, W2)

    p2 = agg128(hs2, src4, dst4)
    hs3 = pl.pallas_call(
        _tc_b, out_shape=jax.ShapeDtypeStruct((_N, _DP3), f32),
    )(p2, hs2, dinv, b2, g2, be2, w3p)

    p3 = _make_agg(_DP3)(hs3, src4, dst4)
    out = pl.pallas_call(
        _tc_c, out_shape=jax.ShapeDtypeStruct((_N, 40), f32),
    )(p3, hs3, dinv, b3)
    return out


# speedup vs baseline: 1.9865x; 1.1199x over previous
import functools

import jax
import jax.numpy as jnp
from jax import lax
from jax.experimental import pallas as pl
from jax.experimental.pallas import tpu as pltpu
from jax.experimental.pallas import tpu_sc as plsc

_N = 10000
_E = 320000
_NC, _NS = 2, 16
_CH = 125
_NCH = 80
_NCH2 = 40
_STRIPE = _N // _NS
_DH = 128
_DP3 = 48


def _sc_mesh():
    return plsc.VectorSubcoreMesh(
        core_axis_name="c", subcore_axis_name="s",
        num_cores=_NC, num_subcores=_NS)


_DEGPAD = 10240
_DSTRIPE = _DEGPAD // _NS


def _make_deg():
    @functools.partial(
        pl.kernel,
        out_type=jax.ShapeDtypeStruct((_NC, _DEGPAD), jnp.float32),
        mesh=_sc_mesh(),
        scratch_types=[
            pltpu.VMEM((_NCH, _CH), jnp.int32),
            pltpu.VMEM((_DSTRIPE,), jnp.float32),
            pltpu.VMEM((_CH,), jnp.float32),
            pltpu.VMEM_SHARED((_DEGPAD,), jnp.float32),
        ],
    )
    def deg_kernel(dst_hbm, out_hbm, dst_v, zbuf, ones_v, acc):
        c = lax.axis_index("c")
        s = lax.axis_index("s")
        zero16 = jnp.zeros((16,), jnp.float32)
        one16 = jnp.ones((16,), jnp.float32)
        for i in range(0, _DSTRIPE, 16):
            zbuf[pl.ds(i, 16)] = zero16
        for i in range(0, _CH - 15, 16):
            ones_v[pl.ds(i, 16)] = one16
        ones_v[pl.ds(_CH - 16, 16)] = one16
        pltpu.sync_copy(zbuf, acc.at[pl.ds(s * _DSTRIPE, _DSTRIPE)])
        plsc.subcore_barrier()
        pltpu.sync_copy(dst_hbm.at[c, s], dst_v)

        def body(j, carry):
            pltpu.sync_copy(ones_v, acc.at[dst_v.at[j]], add=True)
            return carry

        lax.fori_loop(0, _NCH, body, 0)
        plsc.subcore_barrier()
        pltpu.sync_copy(acc.at[pl.ds(s * _DSTRIPE, _DSTRIPE)],
                        out_hbm.at[c, pl.ds(s * _DSTRIPE, _DSTRIPE)])

    return deg_kernel


def _make_agg(d):
    @functools.partial(
        pl.kernel,
        out_type=jax.ShapeDtypeStruct((_NC, _N, d), jnp.float32),
        mesh=_sc_mesh(),
        scratch_types=[
            pltpu.VMEM((_NCH2, _CH), jnp.int32),
            pltpu.VMEM((_NCH2, _CH), jnp.int32),
            pltpu.VMEM_SHARED((_N, d), jnp.float32),
            pltpu.SemaphoreType.DMA,
            pltpu.SemaphoreType.DMA,
            pltpu.SemaphoreType.DMA,
            pltpu.SemaphoreType.DMA,
        ],
        compiler_params=pltpu.CompilerParams(use_tc_tiling_on_sc=False),
    )
    def agg_kernel(hs_hbm, src_hbm, dst_hbm, out_hbm, src_v, dst_v,
                   acc, g0, g1, s0, s1):
        pl.run_scoped(
            functools.partial(_agg_body, hs_hbm, src_hbm, dst_hbm, out_hbm,
                              src_v, dst_v, acc, (g0, g1), (s0, s1), d),
            pltpu.VMEM((2 if d == _DH else 1, _CH, d), jnp.float32))

    def _agg_body(hs_hbm, src_hbm, dst_hbm, out_hbm, src_v, dst_v,
                  acc, gsem, ssem, d, rows_v):
        c = lax.axis_index("c")
        s = lax.axis_index("s")
        zero16 = jnp.zeros((16,), jnp.float32)

        def zrow(r, carry):
            for k in range(d // 16):
                rows_v[0, r, pl.ds(k * 16, 16)] = zero16
            return carry

        lax.fori_loop(0, _CH, zrow, 0)
        for o in range(0, _STRIPE, _CH):
            pltpu.sync_copy(rows_v.at[0],
                            acc.at[pl.ds(s * _STRIPE + o, _CH)])
        plsc.subcore_barrier()

        for ph in range(2):
            pltpu.sync_copy(src_hbm.at[c, s, pl.ds(ph * _NCH2, _NCH2)],
                            src_v)
            pltpu.sync_copy(dst_hbm.at[c, s, pl.ds(ph * _NCH2, _NCH2)],
                            dst_v)
            if d == _DH:
                for b in range(2):
                    pltpu.async_copy(hs_hbm.at[src_v.at[b]], rows_v.at[b],
                                     gsem[b])

                def group(g, carry):
                    base = g * 2
                    for b in range(2):
                        pltpu.make_async_copy(hs_hbm.at[src_v.at[base + b]],
                                              rows_v.at[b], gsem[b]).wait()
                        pltpu.async_copy(rows_v.at[b],
                                         acc.at[dst_v.at[base + b]],
                                         ssem[b], add=True)
                    for b in range(2):
                        pltpu.make_async_copy(rows_v.at[b],
                                              acc.at[dst_v.at[base + b]],
                                              ssem[b]).wait()
                        nxt = base + 2 + b

                        @pl.when(nxt < _NCH2)
                        def _():
                            pltpu.async_copy(hs_hbm.at[src_v.at[nxt]],
                                             rows_v.at[b], gsem[b])
                    return carry

                lax.fori_loop(0, _NCH2 // 2, group, 0)
            else:
                def body(j, carry):
                    pltpu.sync_copy(hs_hbm.at[src_v.at[j]], rows_v.at[0])
                    pltpu.sync_copy(rows_v.at[0], acc.at[dst_v.at[j]],
                                    add=True)
                    return carry

                lax.fori_loop(0, _NCH2, body, 0)
        plsc.subcore_barrier()

        def wb(j, carry):
            o = s * _STRIPE + j * _CH
            pltpu.sync_copy(acc.at[pl.ds(o, _CH)],
                            out_hbm.at[c, pl.ds(o, _CH)])
            return carry

        lax.fori_loop(0, _STRIPE // _CH, wb, 0)

    return agg_kernel


def _tc_a(x_ref, w_ref, degt_ref, hs_ref, dinv_ref):
    deg = jnp.sum(degt_ref[...], axis=1, keepdims=True)[:_N] + 1.0
    dinv = lax.rsqrt(deg)
    h = jnp.dot(x_ref[...], w_ref[...], preferred_element_type=jnp.float32)
    hs_ref[...] = h * dinv
    dinv_ref[...] = dinv


def _tc_b(p_ref, hs_ref, dinv_ref, b_ref, g_ref, be_ref, w_ref, out_ref):
    dinv = dinv_ref[...]
    t = (p_ref[0] + p_ref[1] + hs_ref[...]) * dinv + b_ref[...]
    m = jnp.sum(t, axis=0, keepdims=True) * (1.0 / _N)
    v = jnp.sum(t * t, axis=0, keepdims=True) * (1.0 / _N) - m * m
    a = (t - m) * lax.rsqrt(v + 1e-5) * g_ref[...] + be_ref[...]
    a = jnp.maximum(a, 0.0)
    h = jnp.dot(a, w_ref[...], preferred_element_type=jnp.float32)
    out_ref[...] = h * dinv


def _tc_c(p_ref, hs_ref, dinv_ref, b_ref, out_ref):
    t = (p_ref[0] + p_ref[1] + hs_ref[...]) * dinv_ref[...]
    t = t[:, :40] + b_ref[...]
    mx = jnp.max(t, axis=1, keepdims=True)
    lse = jnp.log(jnp.sum(jnp.exp(t - mx), axis=1, keepdims=True)) + mx
    out_ref[...] = t - lse


def kernel(x, W1, b1, g1, be1, W2, b2, g2, be2, W3, b3, edge_index):
    e5 = edge_index.reshape(2, _NC, _NS, _NCH, _CH)
    src4, dst4 = e5[0], e5[1]
    w3p = jnp.pad(W3, ((0, 0), (0, _DP3 - 40)))

    degp = _make_deg()(dst4)
    degt = degp.T

    f32 = jnp.float32
    hs1, dinv = pl.pallas_call(
        _tc_a,
        out_shape=[jax.ShapeDtypeStruct((_N, _DH), f32),
                   jax.ShapeDtypeStruct((_N, 1), f32)],
    )(x, W1, degt)

    agg128 = _make_agg(_DH)
    p1 = agg128(hs1, src4, dst4)
    hs2 = pl.pallas_call(
        _tc_b, out_shape=jax.ShapeDtypeStruct((_N, _DH), f32),
    )(p1, hs1, dinv, b1, g1, be1, W2)

    p2 = agg128(hs2, src4, dst4)
    hs3 = pl.pallas_call(
        _tc_b, out_shape=jax.ShapeDtypeStruct((_N, _DP3), f32),
    )(p2, hs2, dinv, b2, g2, be2, w3p)

    p3 = _make_agg(_DP3)(hs3, src4, dst4)
    out = pl.pallas_call(
        _tc_c, out_shape=jax.ShapeDtypeStruct((_N, 40), f32),
    )(p3, hs3, dinv, b3)
    return out


# speedup vs baseline: 2.1380x; 1.0763x over previous
import functools

import jax
import jax.numpy as jnp
from jax import lax
from jax.experimental import pallas as pl
from jax.experimental.pallas import tpu as pltpu
from jax.experimental.pallas import tpu_sc as plsc

_N = 10000
_E = 320000
_NC, _NS = 2, 16
_CH = 125
_NCH = 80
_NCH2 = 40
_STRIPE = _N // _NS
_DH = 128
_DP3 = 48


def _sc_mesh():
    return plsc.VectorSubcoreMesh(
        core_axis_name="c", subcore_axis_name="s",
        num_cores=_NC, num_subcores=_NS)


_DEGPAD = 10240
_DSTRIPE = _DEGPAD // _NS


def _make_deg():
    @functools.partial(
        pl.kernel,
        out_type=jax.ShapeDtypeStruct((_NC, _DEGPAD), jnp.float32),
        mesh=_sc_mesh(),
        scratch_types=[
            pltpu.VMEM((_NCH, _CH), jnp.int32),
            pltpu.VMEM((_DSTRIPE,), jnp.float32),
            pltpu.VMEM((_CH,), jnp.float32),
            pltpu.VMEM_SHARED((_DEGPAD,), jnp.float32),
        ],
    )
    def deg_kernel(dst_hbm, out_hbm, dst_v, zbuf, ones_v, acc):
        c = lax.axis_index("c")
        s = lax.axis_index("s")
        zero16 = jnp.zeros((16,), jnp.float32)
        one16 = jnp.ones((16,), jnp.float32)
        for i in range(0, _DSTRIPE, 16):
            zbuf[pl.ds(i, 16)] = zero16
        for i in range(0, _CH - 15, 16):
            ones_v[pl.ds(i, 16)] = one16
        ones_v[pl.ds(_CH - 16, 16)] = one16
        pltpu.sync_copy(zbuf, acc.at[pl.ds(s * _DSTRIPE, _DSTRIPE)])
        plsc.subcore_barrier()
        pltpu.sync_copy(dst_hbm.at[c, s], dst_v)

        def body(j, carry):
            pltpu.sync_copy(ones_v, acc.at[dst_v.at[j]], add=True)
            return carry

        lax.fori_loop(0, _NCH, body, 0)
        plsc.subcore_barrier()
        pltpu.sync_copy(acc.at[pl.ds(s * _DSTRIPE, _DSTRIPE)],
                        out_hbm.at[c, pl.ds(s * _DSTRIPE, _DSTRIPE)])

    return deg_kernel


def _make_agg(d):
    @functools.partial(
        pl.kernel,
        out_type=jax.ShapeDtypeStruct((_NC, _N, d), jnp.float32),
        mesh=_sc_mesh(),
        scratch_types=[
            pltpu.VMEM((_NCH2, _CH), jnp.int32),
            pltpu.VMEM((_NCH2, _CH), jnp.int32),
            pltpu.VMEM_SHARED((_N, d), jnp.float32),
            pltpu.SemaphoreType.DMA,
            pltpu.SemaphoreType.DMA,
            pltpu.SemaphoreType.DMA,
            pltpu.SemaphoreType.DMA,
        ],
        compiler_params=pltpu.CompilerParams(use_tc_tiling_on_sc=False),
    )
    def agg_kernel(hs_hbm, src_hbm, dst_hbm, out_hbm, src_v, dst_v,
                   acc, g0, g1, s0, s1):
        pl.run_scoped(
            functools.partial(_agg_body, hs_hbm, src_hbm, dst_hbm, out_hbm,
                              src_v, dst_v, acc, (g0, g1), (s0, s1), d),
            pltpu.VMEM((2, _CH, d), jnp.float32))

    def _agg_body(hs_hbm, src_hbm, dst_hbm, out_hbm, src_v, dst_v,
                  acc, gsem, ssem, d, rows_v):
        c = lax.axis_index("c")
        s = lax.axis_index("s")
        zero16 = jnp.zeros((16,), jnp.float32)

        def zrow(r, carry):
            for k in range(d // 16):
                rows_v[0, r, pl.ds(k * 16, 16)] = zero16
            return carry

        lax.fori_loop(0, _CH, zrow, 0)
        for o in range(0, _STRIPE, _CH):
            pltpu.sync_copy(rows_v.at[0],
                            acc.at[pl.ds(s * _STRIPE + o, _CH)])
        plsc.subcore_barrier()

        for ph in range(2):
            pltpu.sync_copy(src_hbm.at[c, s, pl.ds(ph * _NCH2, _NCH2)],
                            src_v)
            pltpu.sync_copy(dst_hbm.at[c, s, pl.ds(ph * _NCH2, _NCH2)],
                            dst_v)
            if True:
                for b in range(2):
                    pltpu.async_copy(hs_hbm.at[src_v.at[b]], rows_v.at[b],
                                     gsem[b])

                def group(g, carry):
                    base = g * 2
                    for b in range(2):
                        pltpu.make_async_copy(hs_hbm.at[src_v.at[base + b]],
                                              rows_v.at[b], gsem[b]).wait()
                        pltpu.async_copy(rows_v.at[b],
                                         acc.at[dst_v.at[base + b]],
                                         ssem[b], add=True)
                    for b in range(2):
                        pltpu.make_async_copy(rows_v.at[b],
                                              acc.at[dst_v.at[base + b]],
                                              ssem[b]).wait()
                        nxt = base + 2 + b

                        @pl.when(nxt < _NCH2)
                        def _():
                            pltpu.async_copy(hs_hbm.at[src_v.at[nxt]],
                                             rows_v.at[b], gsem[b])
                    return carry

                lax.fori_loop(0, _NCH2 // 2, group, 0)
            else:
                def body(j, carry):
                    pltpu.sync_copy(hs_hbm.at[src_v.at[j]], rows_v.at[0])
                    pltpu.sync_copy(rows_v.at[0], acc.at[dst_v.at[j]],
                                    add=True)
                    return carry

                lax.fori_loop(0, _NCH2, body, 0)
        plsc.subcore_barrier()

        def wb(j, carry):
            o = s * _STRIPE + j * _CH
            pltpu.sync_copy(acc.at[pl.ds(o, _CH)],
                            out_hbm.at[c, pl.ds(o, _CH)])
            return carry

        lax.fori_loop(0, _STRIPE // _CH, wb, 0)

    return agg_kernel


def _tc_a(x_ref, w_ref, degt_ref, hs_ref, dinv_ref):
    deg = jnp.sum(degt_ref[...], axis=1, keepdims=True)[:_N] + 1.0
    dinv = lax.rsqrt(deg)
    h = jnp.dot(x_ref[...], w_ref[...], preferred_element_type=jnp.float32)
    hs_ref[...] = h * dinv
    dinv_ref[...] = dinv


def _tc_b(p_ref, hs_ref, dinv_ref, b_ref, g_ref, be_ref, w_ref, out_ref):
    dinv = dinv_ref[...]
    t = (p_ref[0] + p_ref[1] + hs_ref[...]) * dinv + b_ref[...]
    m = jnp.sum(t, axis=0, keepdims=True) * (1.0 / _N)
    v = jnp.sum(t * t, axis=0, keepdims=True) * (1.0 / _N) - m * m
    a = (t - m) * lax.rsqrt(v + 1e-5) * g_ref[...] + be_ref[...]
    a = jnp.maximum(a, 0.0)
    h = jnp.dot(a, w_ref[...], preferred_element_type=jnp.float32)
    out_ref[...] = h * dinv


def _tc_c(p_ref, hs_ref, dinv_ref, b_ref, out_ref):
    t = (p_ref[0] + p_ref[1] + hs_ref[...]) * dinv_ref[...]
    t = t[:, :40] + b_ref[...]
    mx = jnp.max(t, axis=1, keepdims=True)
    lse = jnp.log(jnp.sum(jnp.exp(t - mx), axis=1, keepdims=True)) + mx
    out_ref[...] = t - lse


def kernel(x, W1, b1, g1, be1, W2, b2, g2, be2, W3, b3, edge_index):
    e5 = edge_index.reshape(2, _NC, _NS, _NCH, _CH)
    src4, dst4 = e5[0], e5[1]
    w3p = jnp.pad(W3, ((0, 0), (0, _DP3 - 40)))

    degp = _make_deg()(dst4)
    degt = degp.T

    f32 = jnp.float32
    hs1, dinv = pl.pallas_call(
        _tc_a,
        out_shape=[jax.ShapeDtypeStruct((_N, _DH), f32),
                   jax.ShapeDtypeStruct((_N, 1), f32)],
    )(x, W1, degt)

    agg128 = _make_agg(_DH)
    p1 = agg128(hs1, src4, dst4)
    hs2 = pl.pallas_call(
        _tc_b, out_shape=jax.ShapeDtypeStruct((_N, _DH), f32),
    )(p1, hs1, dinv, b1, g1, be1, W2)

    p2 = agg128(hs2, src4, dst4)
    hs3 = pl.pallas_call(
        _tc_b, out_shape=jax.ShapeDtypeStruct((_N, _DP3), f32),
    )(p2, hs2, dinv, b2, g2, be2, w3p)

    p3 = _make_agg(_DP3)(hs3, src4, dst4)
    out = pl.pallas_call(
        _tc_c, out_shape=jax.ShapeDtypeStruct((_N, 40), f32),
    )(p3, hs3, dinv, b3)
    return out


# speedup vs baseline: 2.6804x; 1.2537x over previous
import functools

import jax
import jax.numpy as jnp
from jax import lax
from jax.experimental import pallas as pl
from jax.experimental.pallas import tpu as pltpu
from jax.experimental.pallas import tpu_sc as plsc

_N = 10000
_E = 320000
_NC, _NS = 2, 16
_CH = 125
_NCH = 80
_NCH2 = 40
_NSLAB = _NC * _NS * _NCH
_STRIPE = _N // _NS
_DH = 128
_DP3 = 48

_DEGPAD = 10240
_DSTRIPE = _DEGPAD // _NS


def _sc_mesh():
    return plsc.VectorSubcoreMesh(
        core_axis_name="c", subcore_axis_name="s",
        num_cores=_NC, num_subcores=_NS)


def _make_deg():
    @functools.partial(
        pl.kernel,
        out_type=jax.ShapeDtypeStruct((_NC, _DEGPAD), jnp.float32),
        mesh=_sc_mesh(),
        scratch_types=[
            pltpu.VMEM((_NCH, _CH), jnp.int32),
            pltpu.VMEM((_DSTRIPE,), jnp.float32),
            pltpu.VMEM((_CH,), jnp.float32),
            pltpu.VMEM_SHARED((_DEGPAD,), jnp.float32),
        ],
    )
    def deg_kernel(e_hbm, out_hbm, dst_v, zbuf, ones_v, acc):
        c = lax.axis_index("c")
        s = lax.axis_index("s")
        zero16 = jnp.zeros((16,), jnp.float32)
        one16 = jnp.ones((16,), jnp.float32)
        for i in range(0, _DSTRIPE, 16):
            zbuf[pl.ds(i, 16)] = zero16
        for i in range(0, _CH - 15, 16):
            ones_v[pl.ds(i, 16)] = one16
        ones_v[pl.ds(_CH - 16, 16)] = one16
        pltpu.sync_copy(zbuf, acc.at[pl.ds(s * _DSTRIPE, _DSTRIPE)])
        plsc.subcore_barrier()
        base = (c * _NS + s) * _NCH
        pltpu.sync_copy(e_hbm.at[1, pl.ds(base, _NCH)], dst_v)

        def body(j, carry):
            pltpu.sync_copy(ones_v, acc.at[dst_v.at[j]], add=True)
            return carry

        lax.fori_loop(0, _NCH, body, 0)
        plsc.subcore_barrier()
        pltpu.sync_copy(acc.at[pl.ds(s * _DSTRIPE, _DSTRIPE)],
                        out_hbm.at[c, pl.ds(s * _DSTRIPE, _DSTRIPE)])

    return deg_kernel


def _make_agg(d):
    nb = 2 if d == _DH else 4

    @functools.partial(
        pl.kernel,
        out_type=jax.ShapeDtypeStruct((_NC, _N, d), jnp.float32),
        mesh=_sc_mesh(),
        scratch_types=[
            pltpu.VMEM((_NCH2, _CH), jnp.int32),
            pltpu.VMEM((_NCH2, _CH), jnp.int32),
            pltpu.VMEM_SHARED((_N, d), jnp.float32),
        ] + [pltpu.SemaphoreType.DMA] * (2 * nb),
        compiler_params=pltpu.CompilerParams(use_tc_tiling_on_sc=False),
    )
    def agg_kernel(hs_hbm, e_hbm, out_hbm, src_v, dst_v, acc, *sems):
        pl.run_scoped(
            functools.partial(_agg_body, hs_hbm, e_hbm, out_hbm,
                              src_v, dst_v, acc, sems[:nb], sems[nb:], d),
            pltpu.VMEM((nb, _CH, d), jnp.float32))

    def _agg_body(hs_hbm, e_hbm, out_hbm, src_v, dst_v, acc, gsem, ssem,
                  d, rows_v):
        nbuf = len(gsem)
        c = lax.axis_index("c")
        s = lax.axis_index("s")
        zero16 = jnp.zeros((16,), jnp.float32)

        def zrow(r, carry):
            for k in range(d // 16):
                rows_v[0, r, pl.ds(k * 16, 16)] = zero16
            return carry

        lax.fori_loop(0, _CH, zrow, 0)
        for o in range(0, _STRIPE, _CH):
            pltpu.sync_copy(rows_v.at[0],
                            acc.at[pl.ds(s * _STRIPE + o, _CH)])
        plsc.subcore_barrier()

        base = (c * _NS + s) * _NCH
        for ph in range(2):
            pltpu.sync_copy(e_hbm.at[0, pl.ds(base + ph * _NCH2, _NCH2)],
                            src_v)
            pltpu.sync_copy(e_hbm.at[1, pl.ds(base + ph * _NCH2, _NCH2)],
                            dst_v)
            for b in range(nbuf):
                pltpu.async_copy(hs_hbm.at[src_v.at[b]], rows_v.at[b],
                                 gsem[b])

            def group(g, carry):
                gb = g * nbuf
                for b in range(nbuf):
                    pltpu.make_async_copy(hs_hbm.at[src_v.at[gb + b]],
                                          rows_v.at[b], gsem[b]).wait()
                    pltpu.async_copy(rows_v.at[b],
                                     acc.at[dst_v.at[gb + b]],
                                     ssem[b], add=True)
                for b in range(nbuf):
                    pltpu.make_async_copy(rows_v.at[b],
                                          acc.at[dst_v.at[gb + b]],
                                          ssem[b]).wait()
                    nxt = gb + nbuf + b

                    @pl.when(nxt < _NCH2)
                    def _():
                        pltpu.async_copy(hs_hbm.at[src_v.at[nxt]],
                                         rows_v.at[b], gsem[b])
                return carry

            lax.fori_loop(0, _NCH2 // nbuf, group, 0)
        plsc.subcore_barrier()

        def wb(j, carry):
            o = s * _STRIPE + j * _CH
            pltpu.sync_copy(acc.at[pl.ds(o, _CH)],
                            out_hbm.at[c, pl.ds(o, _CH)])
            return carry

        lax.fori_loop(0, _STRIPE // _CH, wb, 0)

    return agg_kernel


def _tc_a(x_ref, w_ref, degt_ref, hs_ref, dinv_ref):
    deg = jnp.sum(degt_ref[...], axis=1, keepdims=True)[:_N] + 1.0
    dinv = lax.rsqrt(deg)
    h = jnp.dot(x_ref[...], w_ref[...], preferred_element_type=jnp.float32)
    hs_ref[...] = h * dinv
    dinv_ref[...] = dinv


def _tc_b(p_ref, hs_ref, dinv_ref, b_ref, g_ref, be_ref, w_ref, out_ref):
    dinv = dinv_ref[...]
    t = (p_ref[0] + p_ref[1] + hs_ref[...]) * dinv + b_ref[...]
    m = jnp.sum(t, axis=0, keepdims=True) * (1.0 / _N)
    v = jnp.sum(t * t, axis=0, keepdims=True) * (1.0 / _N) - m * m
    a = (t - m) * lax.rsqrt(v + 1e-5) * g_ref[...] + be_ref[...]
    a = jnp.maximum(a, 0.0)
    h = jnp.dot(a, w_ref[...], preferred_element_type=jnp.float32)
    out_ref[...] = h * dinv


def _tc_c(p_ref, hs_ref, dinv_ref, b_ref, out_ref):
    t = (p_ref[0] + p_ref[1] + hs_ref[...]) * dinv_ref[...]
    t = t[:, :40] + b_ref[...]
    mx = jnp.max(t, axis=1, keepdims=True)
    lse = jnp.log(jnp.sum(jnp.exp(t - mx), axis=1, keepdims=True)) + mx
    out_ref[...] = t - lse


def kernel(x, W1, b1, g1, be1, W2, b2, g2, be2, W3, b3, edge_index):
    e3 = edge_index.reshape(2, _NSLAB, _CH)
    w3p = jnp.pad(W3, ((0, 0), (0, _DP3 - 40)))

    degp = _make_deg()(e3)
    degt = degp.T

    f32 = jnp.float32
    hs1, dinv = pl.pallas_call(
        _tc_a,
        out_shape=[jax.ShapeDtypeStruct((_N, _DH), f32),
                   jax.ShapeDtypeStruct((_N, 1), f32)],
    )(x, W1, degt)

    agg128 = _make_agg(_DH)
    p1 = agg128(hs1, e3)
    hs2 = pl.pallas_call(
        _tc_b, out_shape=jax.ShapeDtypeStruct((_N, _DH), f32),
    )(p1, hs1, dinv, b1, g1, be1, W2)

    p2 = agg128(hs2, e3)
    hs3 = pl.pallas_call(
        _tc_b, out_shape=jax.ShapeDtypeStruct((_N, _DP3), f32),
    )(p2, hs2, dinv, b2, g2, be2, w3p)

    p3 = _make_agg(_DP3)(hs3, e3)
    out = pl.pallas_call(
        _tc_c, out_shape=jax.ShapeDtypeStruct((_N, 40), f32),
    )(p3, hs3, dinv, b3)
    return out
